# Initial kernel scaffold; baseline (speedup 1.0000x reference)
import jax
import jax.numpy as jnp
from jax.experimental import pallas as pl


def kernel(X, edge_index, W):
    raise NotImplementedError("write your pallas kernel here")


# speedup vs baseline: 37.1509x; 37.1509x over previous
import functools

import jax
import jax.numpy as jnp
from jax import lax
from jax.experimental import pallas as pl
from jax.experimental.pallas import tpu as pltpu
from jax.experimental.pallas import tpu_sc as plsc

N = 10000
E = 320000
D = 128

NPAD = 10240
TRASH = N
NC = 2
NS = 16
NW = NC * NS
EPW = E // NW
CH = 80
NCHUNK = EPW // CH
EB = 2000
NB = EPW // EB
STRIPE = NPAD // NS
BR = 1024
GRID = NPAD // BR

_mesh = plsc.VectorSubcoreMesh(core_axis_name="c", subcore_axis_name="s")


def _remap_blocks(row_hbm, col_v, stage_r, rowp_v, sem, base):
    for b in range(NB):
        pltpu.async_copy(
            row_hbm.at[pl.ds(base + b * EB, EB)], stage_r, sem).wait()

        def body(jj, _):
            for k in range(CH // 16):
                off = jj * CH + k * 16
                r = stage_r[pl.ds(off, 16)]
                c = col_v[pl.ds(b * EB + off, 16)]
                rowp_v[b * (EB // CH) + jj, pl.ds(k * 16, 16)] = (
                    jnp.where(r == c, TRASH, r))
            return 0

        lax.fori_loop(0, EB // CH, body, 0)


@functools.partial(
    pl.kernel,
    out_type=jax.ShapeDtypeStruct((NC, NPAD), jnp.float32),
    mesh=_mesh,
    scratch_types=[
        pltpu.VMEM((EPW,), jnp.int32),
        pltpu.VMEM((EB,), jnp.int32),
        pltpu.VMEM((NCHUNK, CH), jnp.int32),
        pltpu.VMEM((CH,), jnp.float32),
        pltpu.VMEM((STRIPE,), jnp.float32),
        pltpu.VMEM_SHARED((NPAD,), jnp.float32),
        pltpu.SemaphoreType.DMA,
    ],
)
def _deg_kernel(row_hbm, col_hbm, out_hbm, col_v, stage_r, rowp_v, ones_v,
                zer_v, deg_sh, sem):
    cid = lax.axis_index("c")
    sid = lax.axis_index("s")
    wid = cid * NS + sid
    base = wid * EPW
    cp_c = pltpu.async_copy(col_hbm.at[pl.ds(base, EPW)], col_v, sem)

    z16 = jnp.zeros((16,), jnp.float32)
    for i in range(STRIPE // 16):
        zer_v[pl.ds(i * 16, 16)] = z16
    o16 = jnp.full((16,), 1.0, jnp.float32)
    for k in range(CH // 16):
        ones_v[pl.ds(k * 16, 16)] = o16
    pltpu.sync_copy(zer_v, deg_sh.at[pl.ds(sid * STRIPE, STRIPE)])

    cp_c.wait()
    _remap_blocks(row_hbm, col_v, stage_r, rowp_v, sem, base)
    plsc.subcore_barrier()

    def scat(j, _):
        pltpu.sync_copy(ones_v, deg_sh.at[rowp_v.at[j]], add=True)
        return 0

    lax.fori_loop(0, NCHUNK, scat, 0)
    plsc.subcore_barrier()
    pltpu.sync_copy(deg_sh.at[pl.ds(sid * STRIPE, STRIPE)],
                    out_hbm.at[cid, pl.ds(sid * STRIPE, STRIPE)])


def _scale_mm_body(degt_ref, x_ref, w_ref, z_ref, db_ref):
    dg = degt_ref[...]
    deg = 1.0 + dg[:, 0:1] + dg[:, 1:2]
    dinv = lax.rsqrt(deg)
    db = jnp.broadcast_to(dinv, (BR, D))
    db_ref[...] = db
    xw = jnp.dot(x_ref[...], w_ref[...],
                 preferred_element_type=jnp.float32,
                 precision=lax.Precision.HIGHEST)
    z_ref[...] = db * xw


def _scale_mm(degt, xp, w):
    return pl.pallas_call(
        _scale_mm_body,
        grid=(GRID,),
        in_specs=[
            pl.BlockSpec((BR, NC), lambda i: (i, 0)),
            pl.BlockSpec((BR, D), lambda i: (i, 0)),
            pl.BlockSpec((D, D), lambda i: (0, 0)),
        ],
        out_specs=[
            pl.BlockSpec((BR, D), lambda i: (i, 0)),
            pl.BlockSpec((BR, D), lambda i: (i, 0)),
        ],
        out_shape=[
            jax.ShapeDtypeStruct((NPAD, D), jnp.float32),
            jax.ShapeDtypeStruct((NPAD, D), jnp.float32),
        ],
    )(degt, xp, w)


@functools.partial(
    pl.kernel,
    out_type=jax.ShapeDtypeStruct((NC, NPAD, D), jnp.float32),
    mesh=_mesh,
    scratch_types=[
        pltpu.VMEM((EPW,), jnp.int32),
        pltpu.VMEM((EB,), jnp.int32),
        pltpu.VMEM((NCHUNK, CH), jnp.int32),
        pltpu.VMEM((CH, D), jnp.float32),
        pltpu.VMEM((CH, D), jnp.float32),
        pltpu.VMEM_SHARED((NPAD, D), jnp.float32),
        pltpu.SemaphoreType.DMA,
        pltpu.SemaphoreType.DMA,
        pltpu.SemaphoreType.DMA,
    ],
)
def _agg_kernel(row_hbm, col_hbm, z_hbm, s_hbm, col_v, stage_r, rowp_v,
                buf0, buf1, agg_sh, semr, sem0, sem1):
    cid = lax.axis_index("c")
    sid = lax.axis_index("s")
    wid = cid * NS + sid
    base = wid * EPW
    cp_c = pltpu.async_copy(col_hbm.at[pl.ds(base, EPW)], col_v, semr)

    pltpu.sync_copy(z_hbm.at[pl.ds(sid * STRIPE, STRIPE)],
                    agg_sh.at[pl.ds(sid * STRIPE, STRIPE)])

    cp_c.wait()
    _remap_blocks(row_hbm, col_v, stage_r, rowp_v, semr, base)
    plsc.subcore_barrier()

    pltpu.async_copy(z_hbm.at[col_v.at[pl.ds(0, CH)]], buf0, sem0)

    def step(i, _):
        j = 2 * i
        pltpu.async_copy(
            z_hbm.at[col_v.at[pl.ds((j + 1) * CH, CH)]], buf1, sem1)
        pltpu.make_async_copy(z_hbm.at[pl.ds(0, CH)], buf0, sem0).wait()
        pltpu.sync_copy(buf0, agg_sh.at[rowp_v.at[j]], add=True)
        pltpu.async_copy(
            z_hbm.at[col_v.at[pl.ds((j + 2) * CH, CH)]], buf0, sem0)
        pltpu.make_async_copy(z_hbm.at[pl.ds(0, CH)], buf1, sem1).wait()
        pltpu.sync_copy(buf1, agg_sh.at[rowp_v.at[j + 1]], add=True)
        return 0

    lax.fori_loop(0, NCHUNK // 2, step, 0)
    pltpu.make_async_copy(z_hbm.at[pl.ds(0, CH)], buf0, sem0).wait()
    pltpu.sync_copy(buf0, agg_sh.at[rowp_v.at[NCHUNK - 1]], add=True)
    plsc.subcore_barrier()
    pltpu.sync_copy(agg_sh.at[pl.ds(sid * STRIPE, STRIPE)],
                    s_hbm.at[cid, pl.ds(sid * STRIPE, STRIPE)])


def _combine_body(s_ref, z_ref, db_ref, o_ref):
    s = s_ref[...]
    o_ref[...] = db_ref[...] * (s[0] + s[1] - z_ref[...])


def _combine(s, z, db):
    return pl.pallas_call(
        _combine_body,
        grid=(GRID,),
        in_specs=[
            pl.BlockSpec((NC, BR, D), lambda i: (0, i, 0)),
            pl.BlockSpec((BR, D), lambda i: (i, 0)),
            pl.BlockSpec((BR, D), lambda i: (i, 0)),
        ],
        out_specs=pl.BlockSpec((BR, D), lambda i: (i, 0)),
        out_shape=jax.ShapeDtypeStruct((NPAD, D), jnp.float32),
    )(s, z, db)


def kernel(X, edge_index, W):
    row = edge_index[0]
    col = edge_index[1]
    xp = jnp.pad(X, ((0, NPAD - N), (0, 0)))
    degp = _deg_kernel(row, col)
    degt = degp.T
    z, db = _scale_mm(degt, xp, W)
    s = _agg_kernel(row, col, z)
    out = _combine(s, z, db)
    return out[:N]


# speedup vs baseline: 37.2311x; 1.0022x over previous
import functools

import jax
import jax.numpy as jnp
from jax import lax
from jax.experimental import pallas as pl
from jax.experimental.pallas import tpu as pltpu
from jax.experimental.pallas import tpu_sc as plsc

N = 10000
E = 320000
D = 128

NPAD = 10240
TRASH = N
NC = 2
NS = 16
NW = NC * NS
EPW = E // NW
CH = 80
NCHUNK = EPW // CH
EB = 2000
NB = EPW // EB
STRIPE = NPAD // NS
BR = 1024
GRID = NPAD // BR

_mesh = plsc.VectorSubcoreMesh(core_axis_name="c", subcore_axis_name="s")


def _remap_blocks(row_hbm, col_v, stage_r, rowp_v, sem, base):
    for b in range(NB):
        pltpu.async_copy(
            row_hbm.at[pl.ds(base + b * EB, EB)], stage_r, sem).wait()

        def body(jj, _):
            for k in range(CH // 16):
                off = jj * CH + k * 16
                r = stage_r[pl.ds(off, 16)]
                c = col_v[pl.ds(b * EB + off, 16)]
                rowp_v[b * (EB // CH) + jj, pl.ds(k * 16, 16)] = (
                    jnp.where(r == c, TRASH, r))
            return 0

        lax.fori_loop(0, EB // CH, body, 0)


@functools.partial(
    pl.kernel,
    out_type=jax.ShapeDtypeStruct((NC, NPAD), jnp.float32),
    mesh=_mesh,
    scratch_types=[
        pltpu.VMEM((EPW,), jnp.int32),
        pltpu.VMEM((EB,), jnp.int32),
        pltpu.VMEM((NCHUNK, CH), jnp.int32),
        pltpu.VMEM((CH,), jnp.float32),
        pltpu.VMEM((STRIPE,), jnp.float32),
        pltpu.VMEM_SHARED((NPAD,), jnp.float32),
        pltpu.SemaphoreType.DMA,
    ],
)
def _deg_kernel(row_hbm, col_hbm, out_hbm, col_v, stage_r, rowp_v, ones_v,
                zer_v, deg_sh, sem):
    cid = lax.axis_index("c")
    sid = lax.axis_index("s")
    wid = cid * NS + sid
    base = wid * EPW
    cp_c = pltpu.async_copy(col_hbm.at[pl.ds(base, EPW)], col_v, sem)

    z16 = jnp.zeros((16,), jnp.float32)
    for i in range(STRIPE // 16):
        zer_v[pl.ds(i * 16, 16)] = z16
    o16 = jnp.full((16,), 1.0, jnp.float32)
    for k in range(CH // 16):
        ones_v[pl.ds(k * 16, 16)] = o16
    pltpu.sync_copy(zer_v, deg_sh.at[pl.ds(sid * STRIPE, STRIPE)])

    cp_c.wait()
    _remap_blocks(row_hbm, col_v, stage_r, rowp_v, sem, base)
    plsc.subcore_barrier()

    def scat(j, _):
        pltpu.sync_copy(ones_v, deg_sh.at[rowp_v.at[j]], add=True)
        return 0

    lax.fori_loop(0, NCHUNK, scat, 0)
    plsc.subcore_barrier()
    pltpu.sync_copy(deg_sh.at[pl.ds(sid * STRIPE, STRIPE)],
                    out_hbm.at[cid, pl.ds(sid * STRIPE, STRIPE)])


def _mm_body(x_ref, w_ref, t_ref):
    t_ref[...] = jnp.dot(x_ref[...], w_ref[...],
                         preferred_element_type=jnp.float32,
                         precision=lax.Precision.HIGHEST)


def _mm(xp, w):
    return pl.pallas_call(
        _mm_body,
        grid=(GRID,),
        in_specs=[
            pl.BlockSpec((BR, D), lambda i: (i, 0)),
            pl.BlockSpec((D, D), lambda i: (0, 0)),
        ],
        out_specs=pl.BlockSpec((BR, D), lambda i: (i, 0)),
        out_shape=jax.ShapeDtypeStruct((NPAD, D), jnp.float32),
    )(xp, w)


def _scale_body(degt_ref, t_ref, z_ref, db_ref):
    dg = degt_ref[...]
    deg = 1.0 + dg[:, 0:1] + dg[:, 1:2]
    dinv = lax.rsqrt(deg)
    db = jnp.broadcast_to(dinv, (BR, D))
    db_ref[...] = db
    z_ref[...] = db * t_ref[...]


def _scale(degt, t):
    return pl.pallas_call(
        _scale_body,
        grid=(GRID,),
        in_specs=[
            pl.BlockSpec((BR, NC), lambda i: (i, 0)),
            pl.BlockSpec((BR, D), lambda i: (i, 0)),
        ],
        out_specs=[
            pl.BlockSpec((BR, D), lambda i: (i, 0)),
            pl.BlockSpec((BR, D), lambda i: (i, 0)),
        ],
        out_shape=[
            jax.ShapeDtypeStruct((NPAD, D), jnp.float32),
            jax.ShapeDtypeStruct((NPAD, D), jnp.float32),
        ],
    )(degt, t)


@functools.partial(
    pl.kernel,
    out_type=jax.ShapeDtypeStruct((NC, NPAD, D), jnp.float32),
    mesh=_mesh,
    scratch_types=[
        pltpu.VMEM((EPW,), jnp.int32),
        pltpu.VMEM((EB,), jnp.int32),
        pltpu.VMEM((NCHUNK, CH), jnp.int32),
        pltpu.VMEM((CH, D), jnp.float32),
        pltpu.VMEM((CH, D), jnp.float32),
        pltpu.VMEM_SHARED((NPAD, D), jnp.float32),
        pltpu.SemaphoreType.DMA,
        pltpu.SemaphoreType.DMA,
        pltpu.SemaphoreType.DMA,
    ],
)
def _agg_kernel(row_hbm, col_hbm, z_hbm, s_hbm, col_v, stage_r, rowp_v,
                buf0, buf1, agg_sh, semr, sem0, sem1):
    cid = lax.axis_index("c")
    sid = lax.axis_index("s")
    wid = cid * NS + sid
    base = wid * EPW
    cp_c = pltpu.async_copy(col_hbm.at[pl.ds(base, EPW)], col_v, semr)

    pltpu.sync_copy(z_hbm.at[pl.ds(sid * STRIPE, STRIPE)],
                    agg_sh.at[pl.ds(sid * STRIPE, STRIPE)])

    cp_c.wait()
    _remap_blocks(row_hbm, col_v, stage_r, rowp_v, semr, base)
    plsc.subcore_barrier()

    pltpu.async_copy(z_hbm.at[col_v.at[pl.ds(0, CH)]], buf0, sem0)

    def step(i, _):
        j = 2 * i
        pltpu.async_copy(
            z_hbm.at[col_v.at[pl.ds((j + 1) * CH, CH)]], buf1, sem1)
        pltpu.make_async_copy(z_hbm.at[pl.ds(0, CH)], buf0, sem0).wait()
        pltpu.sync_copy(buf0, agg_sh.at[rowp_v.at[j]], add=True)
        pltpu.async_copy(
            z_hbm.at[col_v.at[pl.ds((j + 2) * CH, CH)]], buf0, sem0)
        pltpu.make_async_copy(z_hbm.at[pl.ds(0, CH)], buf1, sem1).wait()
        pltpu.sync_copy(buf1, agg_sh.at[rowp_v.at[j + 1]], add=True)
        return 0

    lax.fori_loop(0, NCHUNK // 2, step, 0)
    pltpu.make_async_copy(z_hbm.at[pl.ds(0, CH)], buf0, sem0).wait()
    pltpu.sync_copy(buf0, agg_sh.at[rowp_v.at[NCHUNK - 1]], add=True)
    plsc.subcore_barrier()
    pltpu.sync_copy(agg_sh.at[pl.ds(sid * STRIPE, STRIPE)],
                    s_hbm.at[cid, pl.ds(sid * STRIPE, STRIPE)])


def _combine_body(s_ref, z_ref, db_ref, o_ref):
    s = s_ref[...]
    o_ref[...] = db_ref[...] * (s[0] + s[1] - z_ref[...])


def _combine(s, z, db):
    return pl.pallas_call(
        _combine_body,
        grid=(GRID,),
        in_specs=[
            pl.BlockSpec((NC, BR, D), lambda i: (0, i, 0)),
            pl.BlockSpec((BR, D), lambda i: (i, 0)),
            pl.BlockSpec((BR, D), lambda i: (i, 0)),
        ],
        out_specs=pl.BlockSpec((BR, D), lambda i: (i, 0)),
        out_shape=jax.ShapeDtypeStruct((NPAD, D), jnp.float32),
    )(s, z, db)


def kernel(X, edge_index, W):
    row = edge_index[0]
    col = edge_index[1]
    xp = jnp.pad(X, ((0, NPAD - N), (0, 0)))
    t = _mm(xp, W)
    degp = _deg_kernel(row, col)
    degt = degp.T
    z, db = _scale(degt, t)
    s = _agg_kernel(row, col, z)
    out = _combine(s, z, db)
    return out[:N]


# speedup vs baseline: 38.6856x; 1.0391x over previous
import functools

import jax
import jax.numpy as jnp
from jax import lax
from jax.experimental import pallas as pl
from jax.experimental.pallas import tpu as pltpu
from jax.experimental.pallas import tpu_sc as plsc

N = 10000
E = 320000
D = 128

NPAD = 10240
TRASH = N
NC = 2
NS = 16
NW = NC * NS
EPW = E // NW
CH = 80
NCHUNK = EPW // CH
EB = 2000
NB = EPW // EB
STRIPE = NPAD // NS
BR = 1024
GRID = NPAD // BR

_mesh = plsc.VectorSubcoreMesh(core_axis_name="c", subcore_axis_name="s")


def _remap_blocks(row_hbm, col_v, stage_r, rowp_v, sem, base):
    for b in range(NB):
        pltpu.async_copy(
            row_hbm.at[pl.ds(base + b * EB, EB)], stage_r, sem).wait()

        def body(jj, _):
            for k in range(CH // 16):
                off = jj * CH + k * 16
                r = stage_r[pl.ds(off, 16)]
                c = col_v[pl.ds(b * EB + off, 16)]
                rowp_v[b * (EB // CH) + jj, pl.ds(k * 16, 16)] = (
                    jnp.where(r == c, TRASH, r))
            return 0

        lax.fori_loop(0, EB // CH, body, 0)


@functools.partial(
    pl.kernel,
    out_type=jax.ShapeDtypeStruct((NC, NPAD), jnp.float32),
    mesh=_mesh,
    scratch_types=[
        pltpu.VMEM((EPW,), jnp.int32),
        pltpu.VMEM((EB,), jnp.int32),
        pltpu.VMEM((NCHUNK, CH), jnp.int32),
        pltpu.VMEM((CH,), jnp.float32),
        pltpu.VMEM((STRIPE,), jnp.float32),
        pltpu.VMEM_SHARED((NPAD,), jnp.float32),
        pltpu.SemaphoreType.DMA,
    ],
)
def _deg_kernel(row_hbm, col_hbm, out_hbm, col_v, stage_r, rowp_v, ones_v,
                zer_v, deg_sh, sem):
    cid = lax.axis_index("c")
    sid = lax.axis_index("s")
    wid = cid * NS + sid
    base = wid * EPW
    cp_c = pltpu.async_copy(col_hbm.at[pl.ds(base, EPW)], col_v, sem)

    z16 = jnp.zeros((16,), jnp.float32)
    for i in range(STRIPE // 16):
        zer_v[pl.ds(i * 16, 16)] = z16
    o16 = jnp.full((16,), 1.0, jnp.float32)
    for k in range(CH // 16):
        ones_v[pl.ds(k * 16, 16)] = o16
    pltpu.sync_copy(zer_v, deg_sh.at[pl.ds(sid * STRIPE, STRIPE)])

    cp_c.wait()
    _remap_blocks(row_hbm, col_v, stage_r, rowp_v, sem, base)
    plsc.subcore_barrier()

    def scat(j, _):
        pltpu.sync_copy(ones_v, deg_sh.at[rowp_v.at[j]], add=True)
        return 0

    lax.fori_loop(0, NCHUNK, scat, 0)
    plsc.subcore_barrier()
    pltpu.sync_copy(deg_sh.at[pl.ds(sid * STRIPE, STRIPE)],
                    out_hbm.at[cid, pl.ds(sid * STRIPE, STRIPE)])


def _mm_body(x_ref, w_ref, t_ref):
    t_ref[...] = jnp.dot(x_ref[...], w_ref[...],
                         preferred_element_type=jnp.float32,
                         precision=lax.Precision.HIGHEST)


def _mm(xp, w):
    return pl.pallas_call(
        _mm_body,
        grid=(GRID,),
        in_specs=[
            pl.BlockSpec((BR, D), lambda i: (i, 0)),
            pl.BlockSpec((D, D), lambda i: (0, 0)),
        ],
        out_specs=pl.BlockSpec((BR, D), lambda i: (i, 0)),
        out_shape=jax.ShapeDtypeStruct((NPAD, D), jnp.float32),
    )(xp, w)


def _scale_body(degt_ref, t_ref, z_ref, db_ref):
    dg = degt_ref[...]
    deg = 1.0 + dg[:, 0:1] + dg[:, 1:2]
    dinv = lax.rsqrt(deg)
    db = jnp.broadcast_to(dinv, (BR, D))
    db_ref[...] = db
    z_ref[...] = db * t_ref[...]


def _scale(degt, t):
    return pl.pallas_call(
        _scale_body,
        grid=(GRID,),
        in_specs=[
            pl.BlockSpec((BR, NC), lambda i: (i, 0)),
            pl.BlockSpec((BR, D), lambda i: (i, 0)),
        ],
        out_specs=[
            pl.BlockSpec((BR, D), lambda i: (i, 0)),
            pl.BlockSpec((BR, D), lambda i: (i, 0)),
        ],
        out_shape=[
            jax.ShapeDtypeStruct((NPAD, D), jnp.float32),
            jax.ShapeDtypeStruct((NPAD, D), jnp.float32),
        ],
    )(degt, t)


PACK_SHIFT = 15
PACK_MASK = (1 << PACK_SHIFT) - 1
EB3 = 400
NRING_C = 3
NRING_R = 2


@functools.partial(
    pl.kernel,
    out_type=jax.ShapeDtypeStruct((NC, NPAD, D), jnp.float32),
    mesh=_mesh,
    scratch_types=[
        pltpu.VMEM((EB3,), jnp.int32),
        pltpu.VMEM((EB3,), jnp.int32),
        pltpu.VMEM((NCHUNK, CH), jnp.int32),
        pltpu.VMEM((NRING_C, CH), jnp.int32),
        pltpu.VMEM((NRING_R, CH), jnp.int32),
        pltpu.VMEM((3, CH, D), jnp.float32),
        pltpu.VMEM_SHARED((NPAD, D), jnp.float32),
        pltpu.SemaphoreType.DMA,
        pltpu.SemaphoreType.DMA,
        pltpu.SemaphoreType.DMA,
    ],
)
def _agg_kernel(row_hbm, col_hbm, z_hbm, s_hbm, stage_r, stage_c, packed_v,
                idxc_v, idxr_v, buf_v, agg_sh, semr, sem_g, sem_s):
    cid = lax.axis_index("c")
    sid = lax.axis_index("s")
    wid = cid * NS + sid
    base = wid * EPW

    pltpu.sync_copy(z_hbm.at[pl.ds(sid * STRIPE, STRIPE)],
                    agg_sh.at[pl.ds(sid * STRIPE, STRIPE)])

    for b in range(EPW // EB3):
        cp_r = pltpu.async_copy(
            row_hbm.at[pl.ds(base + b * EB3, EB3)], stage_r, semr)
        cp_c = pltpu.async_copy(
            col_hbm.at[pl.ds(base + b * EB3, EB3)], stage_c, semr)
        cp_r.wait()
        cp_c.wait()

        def rbody(jj, _):
            for k in range(CH // 16):
                off = jj * CH + k * 16
                r = stage_r[pl.ds(off, 16)]
                c = stage_c[pl.ds(off, 16)]
                rp = jnp.where(r == c, TRASH, r)
                packed_v[b * (EB3 // CH) + jj, pl.ds(k * 16, 16)] = (
                    (rp << PACK_SHIFT) | c)
            return 0

        lax.fori_loop(0, EB3 // CH, rbody, 0)
    plsc.subcore_barrier()

    def unpack_cols(j, slot):
        for k in range(CH // 16):
            p = packed_v[j, pl.ds(k * 16, 16)]
            idxc_v[slot, pl.ds(k * 16, 16)] = p & PACK_MASK

    def unpack_rows(j, slot):
        for k in range(CH // 16):
            p = packed_v[j, pl.ds(k * 16, 16)]
            idxr_v[slot, pl.ds(k * 16, 16)] = p >> PACK_SHIFT

    def fire_gather(j, slot):
        pltpu.async_copy(z_hbm.at[idxc_v.at[slot]], buf_v.at[slot], sem_g)

    def drain_gather():
        pltpu.make_async_copy(
            z_hbm.at[pl.ds(0, CH)], buf_v.at[0], sem_g).wait()

    def fire_scatter(slot_buf, slot_idx):
        pltpu.async_copy(buf_v.at[slot_buf],
                         agg_sh.at[idxr_v.at[slot_idx]], sem_s, add=True)

    def drain_scatter():
        pltpu.make_async_copy(
            buf_v.at[0], agg_sh.at[idxr_v.at[0]], sem_s).wait()

    unpack_cols(0, 0)
    fire_gather(0, 0)
    unpack_cols(1, 1)
    fire_gather(1, 1)
    drain_gather()
    unpack_rows(0, 0)
    fire_scatter(0, 0)
    unpack_cols(2, 2)
    fire_gather(2, 2)

    def step(j, _):
        slot = lax.rem(j, 3)
        slot_r = lax.rem(j, 2)
        slot_n = lax.rem(j + 2, 3)
        drain_gather()
        unpack_rows(j, slot_r)
        fire_scatter(slot, slot_r)
        drain_scatter()
        unpack_cols(j + 2, slot_n)
        fire_gather(j + 2, slot_n)
        return 0

    lax.fori_loop(1, NCHUNK - 2, step, 0)
    for j in (NCHUNK - 2, NCHUNK - 1):
        drain_gather()
        unpack_rows(j, j % 2)
        fire_scatter(j % 3, j % 2)
        drain_scatter()
    drain_scatter()
    plsc.subcore_barrier()
    pltpu.sync_copy(agg_sh.at[pl.ds(sid * STRIPE, STRIPE)],
                    s_hbm.at[cid, pl.ds(sid * STRIPE, STRIPE)])


def _combine_body(s_ref, z_ref, db_ref, o_ref):
    s = s_ref[...]
    o_ref[...] = db_ref[...] * (s[0] + s[1] - z_ref[...])


def _combine(s, z, db):
    return pl.pallas_call(
        _combine_body,
        grid=(GRID,),
        in_specs=[
            pl.BlockSpec((NC, BR, D), lambda i: (0, i, 0)),
            pl.BlockSpec((BR, D), lambda i: (i, 0)),
            pl.BlockSpec((BR, D), lambda i: (i, 0)),
        ],
        out_specs=pl.BlockSpec((BR, D), lambda i: (i, 0)),
        out_shape=jax.ShapeDtypeStruct((NPAD, D), jnp.float32),
    )(s, z, db)


def kernel(X, edge_index, W):
    row = edge_index[0]
    col = edge_index[1]
    xp = jnp.pad(X, ((0, NPAD - N), (0, 0)))
    t = _mm(xp, W)
    degp = _deg_kernel(row, col)
    degt = degp.T
    z, db = _scale(degt, t)
    s = _agg_kernel(row, col, z)
    out = _combine(s, z, db)
    return out[:N]


# speedup vs baseline: 41.1661x; 1.0641x over previous
import functools

import jax
import jax.numpy as jnp
from jax import lax
from jax.experimental import pallas as pl
from jax.experimental.pallas import tpu as pltpu
from jax.experimental.pallas import tpu_sc as plsc

N = 10000
E = 320000
D = 128

NPAD = 10240
TRASH = N
NC = 2
NS = 16
NW = NC * NS
EPW = E // NW
CH = 80
NCHUNK = EPW // CH
EB = 2000
NB = EPW // EB
STRIPE = NPAD // NS
BR = 1024
GRID = NPAD // BR

_mesh = plsc.VectorSubcoreMesh(core_axis_name="c", subcore_axis_name="s")


def _remap_blocks(row_hbm, col_v, stage_r, rowp_v, sem, base):
    for b in range(NB):
        pltpu.async_copy(
            row_hbm.at[pl.ds(base + b * EB, EB)], stage_r, sem).wait()

        def body(jj, _):
            for k in range(CH // 16):
                off = jj * CH + k * 16
                r = stage_r[pl.ds(off, 16)]
                c = col_v[pl.ds(b * EB + off, 16)]
                rowp_v[b * (EB // CH) + jj, pl.ds(k * 16, 16)] = (
                    jnp.where(r == c, TRASH, r))
            return 0

        lax.fori_loop(0, EB // CH, body, 0)


@functools.partial(
    pl.kernel,
    out_type=jax.ShapeDtypeStruct((NC, NPAD), jnp.float32),
    mesh=_mesh,
    scratch_types=[
        pltpu.VMEM((EPW,), jnp.int32),
        pltpu.VMEM((EB,), jnp.int32),
        pltpu.VMEM((NCHUNK, CH), jnp.int32),
        pltpu.VMEM((CH,), jnp.float32),
        pltpu.VMEM((STRIPE,), jnp.float32),
        pltpu.VMEM_SHARED((NPAD,), jnp.float32),
        pltpu.SemaphoreType.DMA,
        pltpu.SemaphoreType.DMA,
    ],
)
def _deg_kernel(row_hbm, col_hbm, out_hbm, col_v, stage_r, rowp_v, ones_v,
                zer_v, deg_sh, sem, sem_s):
    cid = lax.axis_index("c")
    sid = lax.axis_index("s")
    wid = cid * NS + sid
    base = wid * EPW
    cp_c = pltpu.async_copy(col_hbm.at[pl.ds(base, EPW)], col_v, sem)

    z16 = jnp.zeros((16,), jnp.float32)
    for i in range(STRIPE // 16):
        zer_v[pl.ds(i * 16, 16)] = z16
    o16 = jnp.full((16,), 1.0, jnp.float32)
    for k in range(CH // 16):
        ones_v[pl.ds(k * 16, 16)] = o16
    pltpu.sync_copy(zer_v, deg_sh.at[pl.ds(sid * STRIPE, STRIPE)])

    cp_c.wait()
    _remap_blocks(row_hbm, col_v, stage_r, rowp_v, sem, base)
    plsc.subcore_barrier()

    def fire(j):
        pltpu.async_copy(ones_v, deg_sh.at[rowp_v.at[j]], sem_s, add=True)

    def drain():
        pltpu.make_async_copy(ones_v, deg_sh.at[rowp_v.at[0]], sem_s).wait()

    for j in range(4):
        fire(j)

    def scat(j, _):
        fire(j)
        drain()
        return 0

    lax.fori_loop(4, NCHUNK, scat, 0)
    for _ in range(4):
        drain()
    plsc.subcore_barrier()
    pltpu.sync_copy(deg_sh.at[pl.ds(sid * STRIPE, STRIPE)],
                    out_hbm.at[cid, pl.ds(sid * STRIPE, STRIPE)])


def _mm_body(x_ref, w_ref, t_ref):
    t_ref[...] = jnp.dot(x_ref[...], w_ref[...],
                         preferred_element_type=jnp.float32,
                         precision=lax.Precision.HIGHEST)


def _mm(xp, w):
    return pl.pallas_call(
        _mm_body,
        grid=(GRID,),
        in_specs=[
            pl.BlockSpec((BR, D), lambda i: (i, 0)),
            pl.BlockSpec((D, D), lambda i: (0, 0)),
        ],
        out_specs=pl.BlockSpec((BR, D), lambda i: (i, 0)),
        out_shape=jax.ShapeDtypeStruct((NPAD, D), jnp.float32),
    )(xp, w)


def _scale_body(degt_ref, t_ref, z_ref, db_ref):
    dg = degt_ref[...]
    deg = 1.0 + dg[:, 0:1] + dg[:, 1:2]
    dinv = lax.rsqrt(deg)
    db = jnp.broadcast_to(dinv, (BR, D))
    db_ref[...] = db
    z_ref[...] = db * t_ref[...]


def _scale(degt, t):
    return pl.pallas_call(
        _scale_body,
        grid=(GRID,),
        in_specs=[
            pl.BlockSpec((BR, NC), lambda i: (i, 0)),
            pl.BlockSpec((BR, D), lambda i: (i, 0)),
        ],
        out_specs=[
            pl.BlockSpec((BR, D), lambda i: (i, 0)),
            pl.BlockSpec((BR, D), lambda i: (i, 0)),
        ],
        out_shape=[
            jax.ShapeDtypeStruct((NPAD, D), jnp.float32),
            jax.ShapeDtypeStruct((NPAD, D), jnp.float32),
        ],
    )(degt, t)


PACK_SHIFT = 15
PACK_MASK = (1 << PACK_SHIFT) - 1
EB3 = 400
NRING_C = 3
NRING_R = 2


@functools.partial(
    pl.kernel,
    out_type=jax.ShapeDtypeStruct((NC, NPAD, D), jnp.float32),
    mesh=_mesh,
    scratch_types=[
        pltpu.VMEM((EB3,), jnp.int32),
        pltpu.VMEM((EB3,), jnp.int32),
        pltpu.VMEM((NCHUNK, CH), jnp.int32),
        pltpu.VMEM((NRING_C, CH), jnp.int32),
        pltpu.VMEM((NRING_R, CH), jnp.int32),
        pltpu.VMEM((3, CH, D), jnp.float32),
        pltpu.VMEM_SHARED((NPAD, D), jnp.float32),
        pltpu.SemaphoreType.DMA,
        pltpu.SemaphoreType.DMA,
        pltpu.SemaphoreType.DMA,
    ],
)
def _agg_kernel(row_hbm, col_hbm, z_hbm, s_hbm, stage_r, stage_c, packed_v,
                idxc_v, idxr_v, buf_v, agg_sh, semr, sem_g, sem_s):
    cid = lax.axis_index("c")
    sid = lax.axis_index("s")
    wid = cid * NS + sid
    base = wid * EPW

    pltpu.sync_copy(z_hbm.at[pl.ds(sid * STRIPE, STRIPE)],
                    agg_sh.at[pl.ds(sid * STRIPE, STRIPE)])

    for b in range(EPW // EB3):
        cp_r = pltpu.async_copy(
            row_hbm.at[pl.ds(base + b * EB3, EB3)], stage_r, semr)
        cp_c = pltpu.async_copy(
            col_hbm.at[pl.ds(base + b * EB3, EB3)], stage_c, semr)
        cp_r.wait()
        cp_c.wait()

        def rbody(jj, _):
            for k in range(CH // 16):
                off = jj * CH + k * 16
                r = stage_r[pl.ds(off, 16)]
                c = stage_c[pl.ds(off, 16)]
                rp = jnp.where(r == c, TRASH, r)
                packed_v[b * (EB3 // CH) + jj, pl.ds(k * 16, 16)] = (
                    (rp << PACK_SHIFT) | c)
            return 0

        lax.fori_loop(0, EB3 // CH, rbody, 0)
    plsc.subcore_barrier()

    def unpack_cols(j, slot):
        for k in range(CH // 16):
            p = packed_v[j, pl.ds(k * 16, 16)]
            idxc_v[slot, pl.ds(k * 16, 16)] = p & PACK_MASK

    def unpack_rows(j, slot):
        for k in range(CH // 16):
            p = packed_v[j, pl.ds(k * 16, 16)]
            idxr_v[slot, pl.ds(k * 16, 16)] = p >> PACK_SHIFT

    def fire_gather(j, slot):
        pltpu.async_copy(z_hbm.at[idxc_v.at[slot]], buf_v.at[slot], sem_g)

    def drain_gather():
        pltpu.make_async_copy(
            z_hbm.at[pl.ds(0, CH)], buf_v.at[0], sem_g).wait()

    def fire_scatter(slot_buf, slot_idx):
        pltpu.async_copy(buf_v.at[slot_buf],
                         agg_sh.at[idxr_v.at[slot_idx]], sem_s, add=True)

    def drain_scatter():
        pltpu.make_async_copy(
            buf_v.at[0], agg_sh.at[idxr_v.at[0]], sem_s).wait()

    unpack_cols(0, 0)
    fire_gather(0, 0)
    unpack_cols(1, 1)
    fire_gather(1, 1)
    drain_gather()
    unpack_rows(0, 0)
    fire_scatter(0, 0)
    unpack_cols(2, 2)
    fire_gather(2, 2)

    def step(j, _):
        slot = lax.rem(j, 3)
        slot_r = lax.rem(j, 2)
        slot_n = lax.rem(j + 2, 3)
        drain_gather()
        unpack_rows(j, slot_r)
        fire_scatter(slot, slot_r)
        drain_scatter()
        unpack_cols(j + 2, slot_n)
        fire_gather(j + 2, slot_n)
        return 0

    lax.fori_loop(1, NCHUNK - 2, step, 0)
    for j in (NCHUNK - 2, NCHUNK - 1):
        drain_gather()
        unpack_rows(j, j % 2)
        fire_scatter(j % 3, j % 2)
        drain_scatter()
    drain_scatter()
    plsc.subcore_barrier()
    pltpu.sync_copy(agg_sh.at[pl.ds(sid * STRIPE, STRIPE)],
                    s_hbm.at[cid, pl.ds(sid * STRIPE, STRIPE)])


BR4 = 1000


def _combine_body(s_ref, z_ref, db_ref, o_ref):
    s = s_ref[...]
    o_ref[...] = db_ref[...] * (s[0] + s[1] - z_ref[...])


def _combine(s, z, db):
    return pl.pallas_call(
        _combine_body,
        grid=(N // BR4,),
        in_specs=[
            pl.BlockSpec((NC, BR4, D), lambda i: (0, i, 0)),
            pl.BlockSpec((BR4, D), lambda i: (i, 0)),
            pl.BlockSpec((BR4, D), lambda i: (i, 0)),
        ],
        out_specs=pl.BlockSpec((BR4, D), lambda i: (i, 0)),
        out_shape=jax.ShapeDtypeStruct((N, D), jnp.float32),
    )(s, z, db)


def kernel(X, edge_index, W):
    row = edge_index[0]
    col = edge_index[1]
    t = _mm(X, W)
    degp = _deg_kernel(row, col)
    degt = degp.T
    z, db = _scale(degt, t)
    s = _agg_kernel(row, col, z)
    return _combine(s, z, db)


# speedup vs baseline: 47.7908x; 1.1609x over previous
import functools

import jax
import jax.numpy as jnp
from jax import lax
from jax.experimental import pallas as pl
from jax.experimental.pallas import tpu as pltpu
from jax.experimental.pallas import tpu_sc as plsc

N = 10000
E = 320000
D = 128

NPAD = 10240
TRASH = N
NC = 2
NS = 16
NW = NC * NS
EPW = E // NW
CH = 80
NCHUNK = EPW // CH
EB = 2000
NB = EPW // EB
STRIPE = NPAD // NS
BR = 1024
GRID = NPAD // BR

_mesh = plsc.VectorSubcoreMesh(core_axis_name="c", subcore_axis_name="s")


PACK_SHIFT = 15
PACK_MASK = (1 << PACK_SHIFT) - 1


@functools.partial(
    pl.kernel,
    out_type=[
        jax.ShapeDtypeStruct((NC, NPAD), jnp.float32),
        jax.ShapeDtypeStruct((NW, NCHUNK, CH), jnp.int32),
    ],
    mesh=_mesh,
    scratch_types=[
        pltpu.VMEM((EPW,), jnp.int32),
        pltpu.VMEM((EB,), jnp.int32),
        pltpu.VMEM((NCHUNK, CH), jnp.int32),
        pltpu.VMEM((NCHUNK, CH), jnp.int32),
        pltpu.VMEM((CH,), jnp.float32),
        pltpu.VMEM((STRIPE,), jnp.float32),
        pltpu.VMEM_SHARED((NPAD,), jnp.float32),
        pltpu.SemaphoreType.DMA,
        pltpu.SemaphoreType.DMA,
        pltpu.SemaphoreType.DMA,
    ],
)
def _deg_kernel(flat_hbm, out_hbm, pck_hbm, col_v, stage_r, rowp_v, packed_v,
                ones_v, zer_v, deg_sh, sem, sem_s, sem_w):
    cid = lax.axis_index("c")
    sid = lax.axis_index("s")
    wid = cid * NS + sid
    base = wid * EPW
    cp_c = pltpu.async_copy(flat_hbm.at[pl.ds(E + base, EPW)], col_v, sem)

    z16 = jnp.zeros((16,), jnp.float32)
    for i in range(STRIPE // 16):
        zer_v[pl.ds(i * 16, 16)] = z16
    o16 = jnp.full((16,), 1.0, jnp.float32)
    for k in range(CH // 16):
        ones_v[pl.ds(k * 16, 16)] = o16
    pltpu.sync_copy(zer_v, deg_sh.at[pl.ds(sid * STRIPE, STRIPE)])

    cp_c.wait()
    for b in range(NB):
        pltpu.async_copy(
            flat_hbm.at[pl.ds(base + b * EB, EB)], stage_r, sem).wait()

        def rbody(jj, _):
            for k in range(CH // 16):
                off = jj * CH + k * 16
                r = stage_r[pl.ds(off, 16)]
                c = col_v[pl.ds(b * EB + off, 16)]
                rp = jnp.where(r == c, TRASH, r)
                j = b * (EB // CH) + jj
                rowp_v[j, pl.ds(k * 16, 16)] = rp
                packed_v[j, pl.ds(k * 16, 16)] = (rp << PACK_SHIFT) | c
            return 0

        lax.fori_loop(0, EB // CH, rbody, 0)
    cp_w = pltpu.async_copy(packed_v, pck_hbm.at[wid], sem_w)
    plsc.subcore_barrier()

    def fire(j):
        pltpu.async_copy(ones_v, deg_sh.at[rowp_v.at[j]], sem_s, add=True)

    def drain():
        pltpu.make_async_copy(ones_v, deg_sh.at[rowp_v.at[0]], sem_s).wait()

    for j in range(4):
        fire(j)

    def scat(j, _):
        fire(j)
        drain()
        return 0

    lax.fori_loop(4, NCHUNK, scat, 0)
    for _ in range(4):
        drain()
    cp_w.wait()
    plsc.subcore_barrier()
    pltpu.sync_copy(deg_sh.at[pl.ds(sid * STRIPE, STRIPE)],
                    out_hbm.at[cid, pl.ds(sid * STRIPE, STRIPE)])


def _mm_body(x_ref, w_ref, t_ref):
    t_ref[...] = jnp.dot(x_ref[...], w_ref[...],
                         preferred_element_type=jnp.float32,
                         precision=lax.Precision.HIGHEST)


def _mm(xp, w):
    return pl.pallas_call(
        _mm_body,
        grid=(GRID,),
        in_specs=[
            pl.BlockSpec((BR, D), lambda i: (i, 0)),
            pl.BlockSpec((D, D), lambda i: (0, 0)),
        ],
        out_specs=pl.BlockSpec((BR, D), lambda i: (i, 0)),
        out_shape=jax.ShapeDtypeStruct((NPAD, D), jnp.float32),
    )(xp, w)


def _scale_body(degt_ref, t_ref, z_ref):
    dg = degt_ref[...]
    deg = 1.0 + dg[:, 0:1] + dg[:, 1:2]
    dinv = lax.rsqrt(deg)
    z_ref[...] = dinv * t_ref[...]


def _scale(degt, t):
    return pl.pallas_call(
        _scale_body,
        grid=(GRID,),
        in_specs=[
            pl.BlockSpec((BR, NC), lambda i: (i, 0)),
            pl.BlockSpec((BR, D), lambda i: (i, 0)),
        ],
        out_specs=pl.BlockSpec((BR, D), lambda i: (i, 0)),
        out_shape=jax.ShapeDtypeStruct((NPAD, D), jnp.float32),
    )(degt, t)


NRING_C = 3
NRING_R = 2


@functools.partial(
    pl.kernel,
    out_type=jax.ShapeDtypeStruct((NC, NPAD, D), jnp.float32),
    mesh=_mesh,
    scratch_types=[
        pltpu.VMEM((NCHUNK, CH), jnp.int32),
        pltpu.VMEM((NRING_C, CH), jnp.int32),
        pltpu.VMEM((NRING_R, CH), jnp.int32),
        pltpu.VMEM((3, CH, D), jnp.float32),
        pltpu.VMEM_SHARED((NPAD, D), jnp.float32),
        pltpu.SemaphoreType.DMA,
        pltpu.SemaphoreType.DMA,
        pltpu.SemaphoreType.DMA,
    ],
)
def _agg_kernel(pck_hbm, z_hbm, s_hbm, packed_v,
                idxc_v, idxr_v, buf_v, agg_sh, semr, sem_g, sem_s):
    cid = lax.axis_index("c")
    sid = lax.axis_index("s")
    wid = cid * NS + sid

    cp_p = pltpu.async_copy(pck_hbm.at[wid], packed_v, semr)
    pltpu.sync_copy(z_hbm.at[pl.ds(sid * STRIPE, STRIPE)],
                    agg_sh.at[pl.ds(sid * STRIPE, STRIPE)])
    cp_p.wait()
    plsc.subcore_barrier()

    def unpack_cols(j, slot):
        for k in range(CH // 16):
            p = packed_v[j, pl.ds(k * 16, 16)]
            idxc_v[slot, pl.ds(k * 16, 16)] = p & PACK_MASK

    def unpack_rows(j, slot):
        for k in range(CH // 16):
            p = packed_v[j, pl.ds(k * 16, 16)]
            idxr_v[slot, pl.ds(k * 16, 16)] = p >> PACK_SHIFT

    def fire_gather(j, slot):
        pltpu.async_copy(z_hbm.at[idxc_v.at[slot]], buf_v.at[slot], sem_g)

    def drain_gather():
        pltpu.make_async_copy(
            z_hbm.at[pl.ds(0, CH)], buf_v.at[0], sem_g).wait()

    def fire_scatter(slot_buf, slot_idx):
        pltpu.async_copy(buf_v.at[slot_buf],
                         agg_sh.at[idxr_v.at[slot_idx]], sem_s, add=True)

    def drain_scatter():
        pltpu.make_async_copy(
            buf_v.at[0], agg_sh.at[idxr_v.at[0]], sem_s).wait()

    unpack_cols(0, 0)
    fire_gather(0, 0)
    unpack_cols(1, 1)
    fire_gather(1, 1)
    drain_gather()
    unpack_rows(0, 0)
    fire_scatter(0, 0)
    unpack_cols(2, 2)
    fire_gather(2, 2)

    def step(j, _):
        slot = lax.rem(j, 3)
        slot_r = lax.rem(j, 2)
        slot_n = lax.rem(j + 2, 3)
        drain_gather()
        unpack_rows(j, slot_r)
        fire_scatter(slot, slot_r)
        drain_scatter()
        unpack_cols(j + 2, slot_n)
        fire_gather(j + 2, slot_n)
        return 0

    lax.fori_loop(1, NCHUNK - 2, step, 0)
    for j in (NCHUNK - 2, NCHUNK - 1):
        drain_gather()
        unpack_rows(j, j % 2)
        fire_scatter(j % 3, j % 2)
        drain_scatter()
    drain_scatter()
    plsc.subcore_barrier()
    pltpu.sync_copy(agg_sh.at[pl.ds(sid * STRIPE, STRIPE)],
                    s_hbm.at[cid, pl.ds(sid * STRIPE, STRIPE)])


BR4 = 1000


def _combine_body(s_ref, z_ref, degt_ref, o_ref):
    s = s_ref[...]
    dg = degt_ref[...]
    deg = 1.0 + dg[:, 0:1] + dg[:, 1:2]
    dinv = lax.rsqrt(deg)
    o_ref[...] = dinv * (s[0] + s[1] - z_ref[...])


def _combine(s, z, degt):
    return pl.pallas_call(
        _combine_body,
        grid=(N // BR4,),
        in_specs=[
            pl.BlockSpec((NC, BR4, D), lambda i: (0, i, 0)),
            pl.BlockSpec((BR4, D), lambda i: (i, 0)),
            pl.BlockSpec((BR4, NC), lambda i: (i, 0)),
        ],
        out_specs=pl.BlockSpec((BR4, D), lambda i: (i, 0)),
        out_shape=jax.ShapeDtypeStruct((N, D), jnp.float32),
    )(s, z, degt)


def kernel(X, edge_index, W):
    flat = edge_index.reshape(-1)
    t = _mm(X, W)
    degp, pck = _deg_kernel(flat)
    degt = degp.T
    z = _scale(degt, t)
    s = _agg_kernel(pck, z)
    return _combine(s, z, degt)


# speedup vs baseline: 50.0375x; 1.0470x over previous
import functools

import jax
import jax.numpy as jnp
from jax import lax
from jax.experimental import pallas as pl
from jax.experimental.pallas import tpu as pltpu
from jax.experimental.pallas import tpu_sc as plsc

N = 10000
E = 320000
D = 128

NPAD = 10240
TRASH = N
NC = 2
NS = 16
NW = NC * NS
EPW = E // NW
CH = 80
NCHUNK = EPW // CH
EB = 2000
NB = EPW // EB
STRIPE = NPAD // NS
BR = 2048
GRID = NPAD // BR

_mesh = plsc.VectorSubcoreMesh(core_axis_name="c", subcore_axis_name="s")


PACK_SHIFT = 15
PACK_MASK = (1 << PACK_SHIFT) - 1


@functools.partial(
    pl.kernel,
    out_type=[
        jax.ShapeDtypeStruct((NC, NPAD), jnp.float32),
        jax.ShapeDtypeStruct((NW, NCHUNK, CH), jnp.int32),
    ],
    mesh=_mesh,
    scratch_types=[
        pltpu.VMEM((EB,), jnp.int32),
        pltpu.VMEM((EB,), jnp.int32),
        pltpu.VMEM((EB,), jnp.int32),
        pltpu.VMEM((EB,), jnp.int32),
        pltpu.VMEM((NCHUNK, CH), jnp.int32),
        pltpu.VMEM((NCHUNK, CH), jnp.int32),
        pltpu.VMEM((CH,), jnp.float32),
        pltpu.VMEM((STRIPE,), jnp.float32),
        pltpu.VMEM_SHARED((NPAD,), jnp.float32),
        pltpu.SemaphoreType.DMA,
        pltpu.SemaphoreType.DMA,
        pltpu.SemaphoreType.DMA,
    ],
)
def _deg_kernel(flat_hbm, out_hbm, pck_hbm, stage_r0, stage_r1, stage_c0,
                stage_c1, rowp_v, packed_v, ones_v, zer_v, deg_sh, sem,
                sem_s, sem_w):
    cid = lax.axis_index("c")
    sid = lax.axis_index("s")
    wid = cid * NS + sid
    base = wid * EPW
    stage_r = (stage_r0, stage_r1)
    stage_c = (stage_c0, stage_c1)

    def fetch(b):
        sb = b % 2
        cpr = pltpu.async_copy(
            flat_hbm.at[pl.ds(base + b * EB, EB)], stage_r[sb], sem)
        cpc = pltpu.async_copy(
            flat_hbm.at[pl.ds(E + base + b * EB, EB)], stage_c[sb], sem)
        return cpr, cpc

    cps = fetch(0)

    z16 = jnp.zeros((16,), jnp.float32)
    for i in range(STRIPE // 16):
        zer_v[pl.ds(i * 16, 16)] = z16
    o16 = jnp.full((16,), 1.0, jnp.float32)
    for k in range(CH // 16):
        ones_v[pl.ds(k * 16, 16)] = o16
    pltpu.sync_copy(zer_v, deg_sh.at[pl.ds(sid * STRIPE, STRIPE)])

    for b in range(NB):
        cps[0].wait()
        cps[1].wait()
        if b + 1 < NB:
            cps = fetch(b + 1)
        sb = b % 2

        def rbody(jj, _):
            for k in range(CH // 16):
                off = jj * CH + k * 16
                r = stage_r[sb][pl.ds(off, 16)]
                c = stage_c[sb][pl.ds(off, 16)]
                rp = jnp.where(r == c, TRASH, r)
                j = b * (EB // CH) + jj
                rowp_v[j, pl.ds(k * 16, 16)] = rp
                packed_v[j, pl.ds(k * 16, 16)] = (rp << PACK_SHIFT) | c
            return 0

        lax.fori_loop(0, EB // CH, rbody, 0)
    cp_w = pltpu.async_copy(packed_v, pck_hbm.at[wid], sem_w)
    plsc.subcore_barrier()

    def fire(j):
        pltpu.async_copy(ones_v, deg_sh.at[rowp_v.at[j]], sem_s, add=True)

    def drain():
        pltpu.make_async_copy(ones_v, deg_sh.at[rowp_v.at[0]], sem_s).wait()

    for j in range(4):
        fire(j)

    def scat(j, _):
        fire(j)
        drain()
        return 0

    lax.fori_loop(4, NCHUNK, scat, 0)
    for _ in range(4):
        drain()
    cp_w.wait()
    plsc.subcore_barrier()
    pltpu.sync_copy(deg_sh.at[pl.ds(sid * STRIPE, STRIPE)],
                    out_hbm.at[cid, pl.ds(sid * STRIPE, STRIPE)])


def _mm_body(x_ref, w_ref, t_ref):
    t_ref[...] = jnp.dot(x_ref[...], w_ref[...],
                         preferred_element_type=jnp.float32,
                         precision=lax.Precision.HIGHEST)


def _mm(xp, w):
    return pl.pallas_call(
        _mm_body,
        grid=(GRID,),
        in_specs=[
            pl.BlockSpec((BR, D), lambda i: (i, 0)),
            pl.BlockSpec((D, D), lambda i: (0, 0)),
        ],
        out_specs=pl.BlockSpec((BR, D), lambda i: (i, 0)),
        out_shape=jax.ShapeDtypeStruct((NPAD, D), jnp.float32),
    )(xp, w)


def _scale_body(degt_ref, t_ref, z_ref):
    dg = degt_ref[...]
    deg = 1.0 + dg[:, 0:1] + dg[:, 1:2]
    dinv = lax.rsqrt(deg)
    z_ref[...] = dinv * t_ref[...]


def _scale(degt, t):
    return pl.pallas_call(
        _scale_body,
        grid=(GRID,),
        in_specs=[
            pl.BlockSpec((BR, NC), lambda i: (i, 0)),
            pl.BlockSpec((BR, D), lambda i: (i, 0)),
        ],
        out_specs=pl.BlockSpec((BR, D), lambda i: (i, 0)),
        out_shape=jax.ShapeDtypeStruct((NPAD, D), jnp.float32),
    )(degt, t)


NRING_C = 3
NRING_R = 2


@functools.partial(
    pl.kernel,
    out_type=jax.ShapeDtypeStruct((NC, NPAD, D), jnp.float32),
    mesh=_mesh,
    scratch_types=[
        pltpu.VMEM((NCHUNK, CH), jnp.int32),
        pltpu.VMEM((NRING_C, CH), jnp.int32),
        pltpu.VMEM((NRING_R, CH), jnp.int32),
        pltpu.VMEM((3, CH, D), jnp.float32),
        pltpu.VMEM_SHARED((NPAD, D), jnp.float32),
        pltpu.SemaphoreType.DMA,
        pltpu.SemaphoreType.DMA,
        pltpu.SemaphoreType.DMA,
    ],
)
def _agg_kernel(pck_hbm, z_hbm, s_hbm, packed_v,
                idxc_v, idxr_v, buf_v, agg_sh, semr, sem_g, sem_s):
    cid = lax.axis_index("c")
    sid = lax.axis_index("s")
    wid = cid * NS + sid

    cp_p = pltpu.async_copy(pck_hbm.at[wid], packed_v, semr)
    pltpu.sync_copy(z_hbm.at[pl.ds(sid * STRIPE, STRIPE)],
                    agg_sh.at[pl.ds(sid * STRIPE, STRIPE)])
    cp_p.wait()
    plsc.subcore_barrier()

    def unpack_cols(j, slot):
        for k in range(CH // 16):
            p = packed_v[j, pl.ds(k * 16, 16)]
            idxc_v[slot, pl.ds(k * 16, 16)] = p & PACK_MASK

    def unpack_rows(j, slot):
        for k in range(CH // 16):
            p = packed_v[j, pl.ds(k * 16, 16)]
            idxr_v[slot, pl.ds(k * 16, 16)] = p >> PACK_SHIFT

    def fire_gather(j, slot):
        pltpu.async_copy(z_hbm.at[idxc_v.at[slot]], buf_v.at[slot], sem_g)

    def drain_gather():
        pltpu.make_async_copy(
            z_hbm.at[pl.ds(0, CH)], buf_v.at[0], sem_g).wait()

    def fire_scatter(slot_buf, slot_idx):
        pltpu.async_copy(buf_v.at[slot_buf],
                         agg_sh.at[idxr_v.at[slot_idx]], sem_s, add=True)

    def drain_scatter():
        pltpu.make_async_copy(
            buf_v.at[0], agg_sh.at[idxr_v.at[0]], sem_s).wait()

    unpack_cols(0, 0)
    fire_gather(0, 0)
    unpack_cols(1, 1)
    fire_gather(1, 1)
    drain_gather()
    unpack_rows(0, 0)
    fire_scatter(0, 0)
    unpack_cols(2, 2)
    fire_gather(2, 2)

    def step(j, _):
        slot = lax.rem(j, 3)
        slot_r = lax.rem(j, 2)
        slot_n = lax.rem(j + 2, 3)
        drain_gather()
        unpack_rows(j, slot_r)
        fire_scatter(slot, slot_r)
        drain_scatter()
        unpack_cols(j + 2, slot_n)
        fire_gather(j + 2, slot_n)
        return 0

    lax.fori_loop(1, NCHUNK - 2, step, 0)
    for j in (NCHUNK - 2, NCHUNK - 1):
        drain_gather()
        unpack_rows(j, j % 2)
        fire_scatter(j % 3, j % 2)
        drain_scatter()
    drain_scatter()
    plsc.subcore_barrier()
    pltpu.sync_copy(agg_sh.at[pl.ds(sid * STRIPE, STRIPE)],
                    s_hbm.at[cid, pl.ds(sid * STRIPE, STRIPE)])


BR4 = 2000


def _combine_body(s_ref, z_ref, degt_ref, o_ref):
    s = s_ref[...]
    dg = degt_ref[...]
    deg = 1.0 + dg[:, 0:1] + dg[:, 1:2]
    dinv = lax.rsqrt(deg)
    o_ref[...] = dinv * (s[0] + s[1] - z_ref[...])


def _combine(s, z, degt):
    return pl.pallas_call(
        _combine_body,
        grid=(N // BR4,),
        in_specs=[
            pl.BlockSpec((NC, BR4, D), lambda i: (0, i, 0)),
            pl.BlockSpec((BR4, D), lambda i: (i, 0)),
            pl.BlockSpec((BR4, NC), lambda i: (i, 0)),
        ],
        out_specs=pl.BlockSpec((BR4, D), lambda i: (i, 0)),
        out_shape=jax.ShapeDtypeStruct((N, D), jnp.float32),
    )(s, z, degt)


def kernel(X, edge_index, W):
    flat = edge_index.reshape(-1)
    t = _mm(X, W)
    degp, pck = _deg_kernel(flat)
    degt = degp.T
    z = _scale(degt, t)
    s = _agg_kernel(pck, z)
    return _combine(s, z, degt)


# speedup vs baseline: 50.4164x; 1.0076x over previous
import functools

import jax
import jax.numpy as jnp
from jax import lax
from jax.experimental import pallas as pl
from jax.experimental.pallas import tpu as pltpu
from jax.experimental.pallas import tpu_sc as plsc

N = 10000
E = 320000
D = 128

NPAD = 10240
TRASH = N
NC = 2
NS = 16
NW = NC * NS
EPW = E // NW
CH = 80
NCHUNK = EPW // CH
EB = 2000
NB = EPW // EB
STRIPE = NPAD // NS
BR = 2560
GRID = NPAD // BR

_mesh = plsc.VectorSubcoreMesh(core_axis_name="c", subcore_axis_name="s")


PACK_SHIFT = 15
PACK_MASK = (1 << PACK_SHIFT) - 1


@functools.partial(
    pl.kernel,
    out_type=[
        jax.ShapeDtypeStruct((NC, NPAD), jnp.float32),
        jax.ShapeDtypeStruct((NW, NCHUNK, CH), jnp.int32),
    ],
    mesh=_mesh,
    scratch_types=[
        pltpu.VMEM((EB,), jnp.int32),
        pltpu.VMEM((EB,), jnp.int32),
        pltpu.VMEM((EB,), jnp.int32),
        pltpu.VMEM((EB,), jnp.int32),
        pltpu.VMEM((NCHUNK, CH), jnp.int32),
        pltpu.VMEM((NCHUNK, CH), jnp.int32),
        pltpu.VMEM((CH,), jnp.float32),
        pltpu.VMEM((STRIPE,), jnp.float32),
        pltpu.VMEM_SHARED((NPAD,), jnp.float32),
        pltpu.SemaphoreType.DMA,
        pltpu.SemaphoreType.DMA,
        pltpu.SemaphoreType.DMA,
    ],
)
def _deg_kernel(flat_hbm, out_hbm, pck_hbm, stage_r0, stage_r1, stage_c0,
                stage_c1, rowp_v, packed_v, ones_v, zer_v, deg_sh, sem,
                sem_s, sem_w):
    cid = lax.axis_index("c")
    sid = lax.axis_index("s")
    wid = cid * NS + sid
    base = wid * EPW
    stage_r = (stage_r0, stage_r1)
    stage_c = (stage_c0, stage_c1)

    def fetch(b):
        sb = b % 2
        cpr = pltpu.async_copy(
            flat_hbm.at[pl.ds(base + b * EB, EB)], stage_r[sb], sem)
        cpc = pltpu.async_copy(
            flat_hbm.at[pl.ds(E + base + b * EB, EB)], stage_c[sb], sem)
        return cpr, cpc

    cps = fetch(0)

    z16 = jnp.zeros((16,), jnp.float32)
    for i in range(STRIPE // 16):
        zer_v[pl.ds(i * 16, 16)] = z16
    o16 = jnp.full((16,), 1.0, jnp.float32)
    for k in range(CH // 16):
        ones_v[pl.ds(k * 16, 16)] = o16
    pltpu.sync_copy(zer_v, deg_sh.at[pl.ds(sid * STRIPE, STRIPE)])

    def fire(j):
        pltpu.async_copy(ones_v, deg_sh.at[rowp_v.at[j]], sem_s, add=True)

    def drain():
        pltpu.make_async_copy(ones_v, deg_sh.at[rowp_v.at[0]], sem_s).wait()

    n_fired = 0
    for b in range(NB):
        cps[0].wait()
        cps[1].wait()
        if b + 1 < NB:
            cps = fetch(b + 1)
        sb = b % 2

        def rbody(jj, _):
            for k in range(CH // 16):
                off = jj * CH + k * 16
                r = stage_r[sb][pl.ds(off, 16)]
                c = stage_c[sb][pl.ds(off, 16)]
                rp = jnp.where(r == c, TRASH, r)
                j = b * (EB // CH) + jj
                rowp_v[j, pl.ds(k * 16, 16)] = rp
                packed_v[j, pl.ds(k * 16, 16)] = (rp << PACK_SHIFT) | c
            return 0

        lax.fori_loop(0, EB // CH, rbody, 0)
        if b == 0:
            plsc.subcore_barrier()

        def scat(j, _):
            fire(j)
            drain()
            return 0

        j0 = b * (EB // CH)
        if b == 0:
            for j in range(4):
                fire(j)
            n_fired = 4
        lax.fori_loop(n_fired, j0 + EB // CH, scat, 0)
        n_fired = j0 + EB // CH
    cp_w = pltpu.async_copy(packed_v, pck_hbm.at[wid], sem_w)
    for _ in range(4):
        drain()
    cp_w.wait()
    plsc.subcore_barrier()
    pltpu.sync_copy(deg_sh.at[pl.ds(sid * STRIPE, STRIPE)],
                    out_hbm.at[cid, pl.ds(sid * STRIPE, STRIPE)])


def _mm_body(x_ref, w_ref, t_ref):
    t_ref[...] = jnp.dot(x_ref[...], w_ref[...],
                         preferred_element_type=jnp.float32,
                         precision=lax.Precision.HIGHEST)


def _mm(xp, w):
    return pl.pallas_call(
        _mm_body,
        grid=(GRID,),
        in_specs=[
            pl.BlockSpec((BR, D), lambda i: (i, 0)),
            pl.BlockSpec((D, D), lambda i: (0, 0)),
        ],
        out_specs=pl.BlockSpec((BR, D), lambda i: (i, 0)),
        out_shape=jax.ShapeDtypeStruct((NPAD, D), jnp.float32),
    )(xp, w)


def _scale_body(degt_ref, t_ref, z_ref):
    dg = degt_ref[...]
    deg = 1.0 + dg[:, 0:1] + dg[:, 1:2]
    dinv = lax.rsqrt(deg)
    z_ref[...] = dinv * t_ref[...]


def _scale(degt, t):
    return pl.pallas_call(
        _scale_body,
        grid=(GRID,),
        in_specs=[
            pl.BlockSpec((BR, NC), lambda i: (i, 0)),
            pl.BlockSpec((BR, D), lambda i: (i, 0)),
        ],
        out_specs=pl.BlockSpec((BR, D), lambda i: (i, 0)),
        out_shape=jax.ShapeDtypeStruct((NPAD, D), jnp.float32),
    )(degt, t)


NRING_C = 3
NRING_R = 2


@functools.partial(
    pl.kernel,
    out_type=jax.ShapeDtypeStruct((NC, NPAD, D), jnp.float32),
    mesh=_mesh,
    scratch_types=[
        pltpu.VMEM((NCHUNK, CH), jnp.int32),
        pltpu.VMEM((NRING_C, CH), jnp.int32),
        pltpu.VMEM((NRING_R, CH), jnp.int32),
        pltpu.VMEM((3, CH, D), jnp.float32),
        pltpu.VMEM_SHARED((NPAD, D), jnp.float32),
        pltpu.SemaphoreType.DMA,
        pltpu.SemaphoreType.DMA,
        pltpu.SemaphoreType.DMA,
    ],
)
def _agg_kernel(pck_hbm, z_hbm, s_hbm, packed_v,
                idxc_v, idxr_v, buf_v, agg_sh, semr, sem_g, sem_s):
    cid = lax.axis_index("c")
    sid = lax.axis_index("s")
    wid = cid * NS + sid

    cp_p = pltpu.async_copy(pck_hbm.at[wid], packed_v, semr)
    pltpu.sync_copy(z_hbm.at[pl.ds(sid * STRIPE, STRIPE)],
                    agg_sh.at[pl.ds(sid * STRIPE, STRIPE)])
    cp_p.wait()
    plsc.subcore_barrier()

    def unpack_cols(j, slot):
        for k in range(CH // 16):
            p = packed_v[j, pl.ds(k * 16, 16)]
            idxc_v[slot, pl.ds(k * 16, 16)] = p & PACK_MASK

    def unpack_rows(j, slot):
        for k in range(CH // 16):
            p = packed_v[j, pl.ds(k * 16, 16)]
            idxr_v[slot, pl.ds(k * 16, 16)] = p >> PACK_SHIFT

    def fire_gather(j, slot):
        pltpu.async_copy(z_hbm.at[idxc_v.at[slot]], buf_v.at[slot], sem_g)

    def drain_gather():
        pltpu.make_async_copy(
            z_hbm.at[pl.ds(0, CH)], buf_v.at[0], sem_g).wait()

    def fire_scatter(slot_buf, slot_idx):
        pltpu.async_copy(buf_v.at[slot_buf],
                         agg_sh.at[idxr_v.at[slot_idx]], sem_s, add=True)

    def drain_scatter():
        pltpu.make_async_copy(
            buf_v.at[0], agg_sh.at[idxr_v.at[0]], sem_s).wait()

    unpack_cols(0, 0)
    fire_gather(0, 0)
    unpack_cols(1, 1)
    fire_gather(1, 1)
    drain_gather()
    unpack_rows(0, 0)
    fire_scatter(0, 0)
    unpack_cols(2, 2)
    fire_gather(2, 2)

    def step(j, _):
        slot = lax.rem(j, 3)
        slot_r = lax.rem(j, 2)
        slot_n = lax.rem(j + 2, 3)
        drain_gather()
        unpack_rows(j, slot_r)
        fire_scatter(slot, slot_r)
        drain_scatter()
        unpack_cols(j + 2, slot_n)
        fire_gather(j + 2, slot_n)
        return 0

    lax.fori_loop(1, NCHUNK - 2, step, 0)
    for j in (NCHUNK - 2, NCHUNK - 1):
        drain_gather()
        unpack_rows(j, j % 2)
        fire_scatter(j % 3, j % 2)
        drain_scatter()
    drain_scatter()
    plsc.subcore_barrier()
    pltpu.sync_copy(agg_sh.at[pl.ds(sid * STRIPE, STRIPE)],
                    s_hbm.at[cid, pl.ds(sid * STRIPE, STRIPE)])


BR4 = 2000


def _combine_body(s_ref, z_ref, degt_ref, o_ref):
    s = s_ref[...]
    dg = degt_ref[...]
    deg = 1.0 + dg[:, 0:1] + dg[:, 1:2]
    dinv = lax.rsqrt(deg)
    o_ref[...] = dinv * (s[0] + s[1] - z_ref[...])


def _combine(s, z, degt):
    return pl.pallas_call(
        _combine_body,
        grid=(N // BR4,),
        in_specs=[
            pl.BlockSpec((NC, BR4, D), lambda i: (0, i, 0)),
            pl.BlockSpec((BR4, D), lambda i: (i, 0)),
            pl.BlockSpec((BR4, NC), lambda i: (i, 0)),
        ],
        out_specs=pl.BlockSpec((BR4, D), lambda i: (i, 0)),
        out_shape=jax.ShapeDtypeStruct((N, D), jnp.float32),
    )(s, z, degt)


def kernel(X, edge_index, W):
    flat = edge_index.reshape(-1)
    t = _mm(X, W)
    degp, pck = _deg_kernel(flat)
    degt = degp.T
    z = _scale(degt, t)
    s = _agg_kernel(pck, z)
    return _combine(s, z, degt)
